# Initial kernel scaffold; baseline (speedup 1.0000x reference)
import jax
import jax.numpy as jnp
from jax.experimental import pallas as pl


def kernel(h, e, graph, ew, U_w, U_b, V_w, V_b, A_w, A_b, Bm_w, Bm_b, C_w, C_b, D_w, D_b, U_ew_w, U_ew_b, V_ew_w, V_ew_b, g_h, b_h, g_e, b_e, g_ew, b_ew):
    raise NotImplementedError("write your pallas kernel here")


# speedup vs baseline: 1.9987x; 1.9987x over previous
import jax
import jax.numpy as jnp
from jax.experimental import pallas as pl
from jax.experimental.pallas import tpu as pltpu

B, V, H = 2, 256, 128
R = 8


def _mm(x, w):
    return jax.lax.dot_general(
        x, w, (((1,), (1,)), ((), ())),
        preferred_element_type=jnp.float32,
        precision=jax.lax.Precision.HIGHEST,
    )


def _ln(x, g, b):
    m = jnp.mean(x, axis=-1, keepdims=True)
    v = jnp.mean((x - m) ** 2, axis=-1, keepdims=True)
    return (x - m) * jax.lax.rsqrt(v + 1e-5) * g + b


def _gnn_kernel(h_ref, e_ref, graph_ref, ew_ref,
                U_w_ref, V_w_ref, A_w_ref, Bm_w_ref,
                C_w_ref, D_w_ref, U_ew_w_ref, V_ew_w_ref,
                ub_ref, abd_ref, cb_ref, vb_ref, uewb_ref,
                g_h_ref, b_h_ref, g_e_ref, b_e_ref, g_ew_ref, b_ew_ref,
                h_out_ref, e_out_ref, ew_out_ref,
                uh_s, vh_s, ah_s, bh_s):
    i = pl.program_id(1)

    @pl.when(i == 0)
    def _():
        hb = h_ref[0]
        uh_s[...] = _mm(hb, U_w_ref[...])
        vh_s[...] = _mm(hb, V_w_ref[...])
        ah_s[...] = _mm(hb, A_w_ref[...])
        bh_s[...] = _mm(hb, Bm_w_ref[...])

    ew_blk = ew_ref[0]
    e_blk = e_ref[0]
    rows_ew = ew_blk.reshape(R * V, H)
    rows_e = e_blk.reshape(R * V, H)

    Uew = _mm(rows_ew, U_ew_w_ref[...]).reshape(R, V, H)
    Dew = _mm(rows_ew, D_w_ref[...]).reshape(R, V, H)
    Vew = _mm(rows_ew, V_ew_w_ref[...]).reshape(R, V, H)
    Ce = _mm(rows_e, C_w_ref[...]).reshape(R, V, H)

    g4 = graph_ref[0][:, :, None]
    bh_blk = bh_s[pl.ds(i * R, R), :]
    ew2 = (ah_s[...][None, :, :] + (bh_blk + abd_ref[0])[:, None, :] + Dew) * g4
    e2 = ew2 + (Ce + cb_ref[0]) * g4
    gates = jax.nn.sigmoid(e2)

    vh_tot = (vh_s[...] + vb_ref[0])[None, :, :] + Vew
    agg = jnp.sum(gates * vh_tot * g4, axis=1)

    h2 = uh_s[pl.ds(i * R, R), :] + ub_ref[0] + agg
    h2 = jax.nn.relu(_ln(h2, g_h_ref[0], b_h_ref[0]))
    h_out_ref[0] = h_ref[0, pl.ds(i * R, R), :] + h2

    e2 = jax.nn.relu(_ln(e2, g_e_ref[0], b_e_ref[0]))
    e_out_ref[0] = e_blk + e2

    ew2 = ew2 + Uew + uewb_ref[0]
    ew2 = jax.nn.relu(_ln(ew2, g_ew_ref[0], b_ew_ref[0]))
    ew_out_ref[0] = ew_blk + ew2


@jax.jit
def _run(h, e, graph, ew, U_w, V_w, A_w, Bm_w, C_w, D_w, U_ew_w, V_ew_w,
         ub, abd, cb, vb, uewb, g_h, b_h, g_e, b_e, g_ew, b_ew):
    grid = (B, V // R)
    full_w = pl.BlockSpec((H, H), lambda b, i: (0, 0))
    vec = pl.BlockSpec((1, H), lambda b, i: (0, 0))
    edge = pl.BlockSpec((1, R, V, H), lambda b, i: (b, i, 0, 0))
    return pl.pallas_call(
        _gnn_kernel,
        grid=grid,
        in_specs=[
            pl.BlockSpec((1, V, H), lambda b, i: (b, 0, 0)),
            edge,
            pl.BlockSpec((1, R, V), lambda b, i: (b, i, 0)),
            edge,
            full_w, full_w, full_w, full_w,
            full_w, full_w, full_w, full_w,
            vec, vec, vec, vec, vec,
            vec, vec, vec, vec, vec, vec,
        ],
        out_specs=[
            pl.BlockSpec((1, R, H), lambda b, i: (b, i, 0)),
            edge,
            edge,
        ],
        out_shape=[
            jax.ShapeDtypeStruct((B, V, H), jnp.float32),
            jax.ShapeDtypeStruct((B, V, V, H), jnp.float32),
            jax.ShapeDtypeStruct((B, V, V, H), jnp.float32),
        ],
        scratch_shapes=[pltpu.VMEM((V, H), jnp.float32)] * 4,
        compiler_params=pltpu.CompilerParams(
            dimension_semantics=("arbitrary", "arbitrary"),
        ),
    )(h, e, graph, ew, U_w, V_w, A_w, Bm_w, C_w, D_w, U_ew_w, V_ew_w,
      ub, abd, cb, vb, uewb, g_h, b_h, g_e, b_e, g_ew, b_ew)


def kernel(h, e, graph, ew, U_w, U_b, V_w, V_b, A_w, A_b, Bm_w, Bm_b,
           C_w, C_b, D_w, D_b, U_ew_w, U_ew_b, V_ew_w, V_ew_b,
           g_h, b_h, g_e, b_e, g_ew, b_ew):
    r = lambda x: x.reshape(1, H)
    ub = r(U_b)
    abd = r(A_b + Bm_b + D_b)
    cb = r(C_b)
    vb = r(V_b + V_ew_b)
    uewb = r(U_ew_b)
    return _run(h, e, graph, ew, U_w, V_w, A_w, Bm_w, C_w, D_w, U_ew_w,
                V_ew_w, ub, abd, cb, vb, uewb,
                r(g_h), r(b_h), r(g_e), r(b_e), r(g_ew), r(b_ew))


# speedup vs baseline: 3.9137x; 1.9582x over previous
import jax
import jax.numpy as jnp
from jax.experimental import pallas as pl
from jax.experimental.pallas import tpu as pltpu

B, V, H = 2, 256, 128
R = 8


def _mm(x, w):
    return jax.lax.dot_general(
        x, w, (((1,), (1,)), ((), ())),
        preferred_element_type=jnp.float32,
        precision=jax.lax.Precision.DEFAULT,
    )


def _ln(x, g, b):
    m = jnp.mean(x, axis=-1, keepdims=True)
    v = jnp.mean((x - m) ** 2, axis=-1, keepdims=True)
    return (x - m) * jax.lax.rsqrt(v + 1e-5) * g + b


def _gnn_kernel(h_ref, e_ref, graph_ref, ew_ref,
                U_w_ref, V_w_ref, A_w_ref, Bm_w_ref,
                C_w_ref, D_w_ref, U_ew_w_ref, V_ew_w_ref,
                ub_ref, abd_ref, cb_ref, vb_ref, uewb_ref,
                g_h_ref, b_h_ref, g_e_ref, b_e_ref, g_ew_ref, b_ew_ref,
                h_out_ref, e_out_ref, ew_out_ref,
                uh_s, vh_s, ah_s, bh_s):
    i = pl.program_id(1)

    @pl.when(i == 0)
    def _():
        hb = h_ref[0]
        uh_s[...] = _mm(hb, U_w_ref[...])
        vh_s[...] = _mm(hb, V_w_ref[...])
        ah_s[...] = _mm(hb, A_w_ref[...])
        bh_s[...] = _mm(hb, Bm_w_ref[...])

    ew_blk = ew_ref[0]
    e_blk = e_ref[0]
    rows_ew = ew_blk.reshape(R * V, H)
    rows_e = e_blk.reshape(R * V, H)

    Uew = _mm(rows_ew, U_ew_w_ref[...]).reshape(R, V, H)
    Dew = _mm(rows_ew, D_w_ref[...]).reshape(R, V, H)
    Vew = _mm(rows_ew, V_ew_w_ref[...]).reshape(R, V, H)
    Ce = _mm(rows_e, C_w_ref[...]).reshape(R, V, H)

    g4 = graph_ref[0][:, :, None]
    bh_blk = bh_s[pl.ds(i * R, R), :]
    ew2 = (ah_s[...][None, :, :] + (bh_blk + abd_ref[0])[:, None, :] + Dew) * g4
    e2 = ew2 + (Ce + cb_ref[0]) * g4
    gates = jax.nn.sigmoid(e2)

    vh_tot = (vh_s[...] + vb_ref[0])[None, :, :] + Vew
    agg = jnp.sum(gates * vh_tot * g4, axis=1)

    h2 = uh_s[pl.ds(i * R, R), :] + ub_ref[0] + agg
    h2 = jax.nn.relu(_ln(h2, g_h_ref[0], b_h_ref[0]))
    h_out_ref[0] = h_ref[0, pl.ds(i * R, R), :] + h2

    e2 = jax.nn.relu(_ln(e2, g_e_ref[0], b_e_ref[0]))
    e_out_ref[0] = e_blk + e2

    ew2 = ew2 + Uew + uewb_ref[0]
    ew2 = jax.nn.relu(_ln(ew2, g_ew_ref[0], b_ew_ref[0]))
    ew_out_ref[0] = ew_blk + ew2


@jax.jit
def _run(h, e, graph, ew, U_w, V_w, A_w, Bm_w, C_w, D_w, U_ew_w, V_ew_w,
         ub, abd, cb, vb, uewb, g_h, b_h, g_e, b_e, g_ew, b_ew):
    grid = (B, V // R)
    full_w = pl.BlockSpec((H, H), lambda b, i: (0, 0))
    vec = pl.BlockSpec((1, H), lambda b, i: (0, 0))
    edge = pl.BlockSpec((1, R, V, H), lambda b, i: (b, i, 0, 0))
    return pl.pallas_call(
        _gnn_kernel,
        grid=grid,
        in_specs=[
            pl.BlockSpec((1, V, H), lambda b, i: (b, 0, 0)),
            edge,
            pl.BlockSpec((1, R, V), lambda b, i: (b, i, 0)),
            edge,
            full_w, full_w, full_w, full_w,
            full_w, full_w, full_w, full_w,
            vec, vec, vec, vec, vec,
            vec, vec, vec, vec, vec, vec,
        ],
        out_specs=[
            pl.BlockSpec((1, R, H), lambda b, i: (b, i, 0)),
            edge,
            edge,
        ],
        out_shape=[
            jax.ShapeDtypeStruct((B, V, H), jnp.float32),
            jax.ShapeDtypeStruct((B, V, V, H), jnp.float32),
            jax.ShapeDtypeStruct((B, V, V, H), jnp.float32),
        ],
        scratch_shapes=[pltpu.VMEM((V, H), jnp.float32)] * 4,
        compiler_params=pltpu.CompilerParams(
            dimension_semantics=("arbitrary", "arbitrary"),
        ),
    )(h, e, graph, ew, U_w, V_w, A_w, Bm_w, C_w, D_w, U_ew_w, V_ew_w,
      ub, abd, cb, vb, uewb, g_h, b_h, g_e, b_e, g_ew, b_ew)


def kernel(h, e, graph, ew, U_w, U_b, V_w, V_b, A_w, A_b, Bm_w, Bm_b,
           C_w, C_b, D_w, D_b, U_ew_w, U_ew_b, V_ew_w, V_ew_b,
           g_h, b_h, g_e, b_e, g_ew, b_ew):
    r = lambda x: x.reshape(1, H)
    ub = r(U_b)
    abd = r(A_b + Bm_b + D_b)
    cb = r(C_b)
    vb = r(V_b + V_ew_b)
    uewb = r(U_ew_b)
    return _run(h, e, graph, ew, U_w, V_w, A_w, Bm_w, C_w, D_w, U_ew_w,
                V_ew_w, ub, abd, cb, vb, uewb,
                r(g_h), r(b_h), r(g_e), r(b_e), r(g_ew), r(b_ew))
